# baseline (speedup 1.0000x reference)
import functools

import jax
import jax.numpy as jnp
from jax import lax
from jax.experimental import pallas as pl
from jax.experimental.pallas import tpu as pltpu
from jax.experimental.pallas import tpu_sc as plsc

B, F, V, D = 4096, 26, 100000, 16
H1, H2 = 512, 256

NC, NS = 2, 16
NW = NC * NS
ROWS = B * F
RPW = ROWS // NW
CHUNK = 128
NCHUNK = RPW // CHUNK

BLK = 512


def _sc_gather_body(emb_hbm, bias_hbm, idx_hbm, out_emb, out_bias,
                    idx_v, emb_v, bias_v, sem_i, sem_e, sem_b):
    wid = lax.axis_index("s") * NC + lax.axis_index("c")
    pltpu.make_async_copy(idx_hbm.at[wid], idx_v, sem_i).start()
    pltpu.make_async_copy(idx_hbm.at[wid], idx_v, sem_i).wait()

    def fire(j, carry):
        pltpu.make_async_copy(emb_hbm.at[idx_v.at[j]], emb_v.at[j], sem_e).start()
        pltpu.make_async_copy(bias_hbm.at[idx_v.at[j]], bias_v.at[j], sem_b).start()
        return carry

    lax.fori_loop(0, NCHUNK, fire, 0)

    def drain(j, carry):
        pltpu.make_async_copy(emb_hbm.at[idx_v.at[j]], emb_v.at[j], sem_e).wait()
        pltpu.make_async_copy(bias_hbm.at[idx_v.at[j]], bias_v.at[j], sem_b).wait()
        return carry

    lax.fori_loop(0, NCHUNK, drain, 0)
    pltpu.sync_copy(emb_v, out_emb.at[wid])
    pltpu.sync_copy(bias_v, out_bias.at[wid])


@jax.jit
def _sc_gather(emb2d, bias2d, idx3d):
    mesh = plsc.VectorSubcoreMesh(core_axis_name="c", subcore_axis_name="s")
    return pl.kernel(
        _sc_gather_body,
        out_type=(
            jax.ShapeDtypeStruct((NW, NCHUNK, CHUNK, D), jnp.float32),
            jax.ShapeDtypeStruct((NW, NCHUNK, CHUNK, 1), jnp.float32),
        ),
        mesh=mesh,
        compiler_params=pltpu.CompilerParams(use_tc_tiling_on_sc=False),
        scratch_types=[
            pltpu.VMEM((NCHUNK, CHUNK), jnp.int32),
            pltpu.VMEM((NCHUNK, CHUNK, D), jnp.float32),
            pltpu.VMEM((NCHUNK, CHUNK, 1), jnp.float32),
            pltpu.SemaphoreType.DMA,
            pltpu.SemaphoreType.DMA,
            pltpu.SemaphoreType.DMA,
        ],
    )(emb2d, bias2d, idx3d)


def _mlp_body(x_ref, bias_ref, w1_ref, b1_ref, w2_ref, b2_ref, w3_ref, b3_ref,
              out_ref):
    x = x_ref[...]
    h = jnp.maximum(
        jnp.dot(x, w1_ref[...], preferred_element_type=jnp.float32)
        + b1_ref[...], 0.0)
    h = jnp.maximum(
        jnp.dot(h, w2_ref[...], preferred_element_type=jnp.float32)
        + b2_ref[...], 0.0)
    o = jnp.dot(h, w3_ref[...], preferred_element_type=jnp.float32) + b3_ref[...]
    bsum = jnp.sum(bias_ref[...], axis=1, keepdims=True)
    out_ref[...] = jax.nn.sigmoid(o + bsum)


@jax.jit
def _mlp(x, biasmat, w1, b1, w2, b2, w3, b3):
    grid = (B // BLK,)
    return pl.pallas_call(
        _mlp_body,
        grid=grid,
        in_specs=[
            pl.BlockSpec((BLK, F * D), lambda i: (i, 0)),
            pl.BlockSpec((BLK, F), lambda i: (i, 0)),
            pl.BlockSpec((F * D, H1), lambda i: (0, 0)),
            pl.BlockSpec((1, H1), lambda i: (0, 0)),
            pl.BlockSpec((H1, H2), lambda i: (0, 0)),
            pl.BlockSpec((1, H2), lambda i: (0, 0)),
            pl.BlockSpec((H2, 1), lambda i: (0, 0)),
            pl.BlockSpec((1, 1), lambda i: (0, 0)),
        ],
        out_specs=pl.BlockSpec((BLK, 1), lambda i: (i, 0)),
        out_shape=jax.ShapeDtypeStruct((B, 1), jnp.float32),
    )(x, biasmat, w1, b1, w2, b2, w3, b3)


def kernel(inputs, emb_tables, bias_tables, W1, b1, W2, b2, W3, b3):
    emb2d = emb_tables.reshape(F * V, D)
    bias2d = bias_tables.reshape(F * V, 1)
    flat_idx = (inputs.astype(jnp.int32)
                + (jnp.arange(F, dtype=jnp.int32) * V)[None, :])
    idx3d = flat_idx.reshape(NW, NCHUNK, CHUNK)

    rows, brow = _sc_gather(emb2d, bias2d, idx3d)
    x = rows.reshape(B, F * D)
    biasmat = brow.reshape(B, F)

    out = _mlp(x, biasmat, W1, b1.reshape(1, H1), W2, b2.reshape(1, H2),
               W3, b3.reshape(1, 1))
    return out.reshape(B)


# speedup vs baseline: 32.6172x; 32.6172x over previous
import jax
import jax.numpy as jnp
from jax import lax
from jax.experimental import pallas as pl
from jax.experimental.pallas import tpu as pltpu
from jax.experimental.pallas import tpu_sc as plsc

B, F, V, D = 4096, 26, 100000, 16
H1, H2 = 512, 256

NC, NS = 2, 16
NW = NC * NS
K = F * D
CPW = K // NW

BLK = 512


def _sc_body(emb_hbm, bias_hbm, idx_hbm, xt_hbm, bt_hbm,
             col_v, idx_v, out_v, sem):
    wid = lax.axis_index("s") * NC + lax.axis_index("c")
    k0 = wid * CPW

    def extract(n, _):
        ivec = idx_v[pl.ds(n * 16, 16)]
        out_v[pl.ds(n * 16, 16)] = plsc.load_gather(col_v, [ivec])
        return _

    def do_col(k, carry):
        pltpu.sync_copy(idx_hbm.at[k // D], idx_v)
        pltpu.sync_copy(emb_hbm.at[k], col_v)
        lax.fori_loop(0, B // 16, extract, 0, unroll=8)
        pltpu.sync_copy(out_v, xt_hbm.at[k])
        return carry

    lax.fori_loop(k0, k0 + CPW, do_col, 0)

    @pl.when(wid < F)
    def _():
        pltpu.sync_copy(idx_hbm.at[wid], idx_v)
        pltpu.sync_copy(bias_hbm.at[wid], col_v)
        lax.fori_loop(0, B // 16, extract, 0, unroll=8)
        pltpu.sync_copy(out_v, bt_hbm.at[wid])


def _sc_gather(embT, biasT, idxT):
    mesh = plsc.VectorSubcoreMesh(core_axis_name="c", subcore_axis_name="s")
    return pl.kernel(
        _sc_body,
        out_type=(
            jax.ShapeDtypeStruct((K, B), jnp.float32),
            jax.ShapeDtypeStruct((F, B), jnp.float32),
        ),
        mesh=mesh,
        compiler_params=pltpu.CompilerParams(needs_layout_passes=False),
        scratch_types=[
            pltpu.VMEM((V,), jnp.float32),
            pltpu.VMEM((B,), jnp.int32),
            pltpu.VMEM((B,), jnp.float32),
            pltpu.SemaphoreType.DMA,
        ],
    )(embT, biasT, idxT)


def _mlp_body(xt_ref, bt_ref, w1_ref, b1_ref, w2_ref, b2_ref, w3_ref, b3_ref,
              out_ref):
    xt = xt_ref[...]
    h = jnp.maximum(
        lax.dot_general(xt, w1_ref[...], (((0,), (0,)), ((), ())),
                        preferred_element_type=jnp.float32)
        + b1_ref[...], 0.0)
    h = jnp.maximum(
        jnp.dot(h, w2_ref[...], preferred_element_type=jnp.float32)
        + b2_ref[...], 0.0)
    o = lax.dot_general(w3_ref[...], h, (((0,), (1,)), ((), ())),
                        preferred_element_type=jnp.float32)
    bsum = jnp.sum(bt_ref[...], axis=0, keepdims=True)
    out_ref[...] = jax.nn.sigmoid(o + bsum + b3_ref[...])


def _mlp(xt, bt, w1, b1, w2, b2, w3, b3):
    return pl.pallas_call(
        _mlp_body,
        grid=(B // BLK,),
        in_specs=[
            pl.BlockSpec((K, BLK), lambda i: (0, i)),
            pl.BlockSpec((F, BLK), lambda i: (0, i)),
            pl.BlockSpec((K, H1), lambda i: (0, 0)),
            pl.BlockSpec((1, H1), lambda i: (0, 0)),
            pl.BlockSpec((H1, H2), lambda i: (0, 0)),
            pl.BlockSpec((1, H2), lambda i: (0, 0)),
            pl.BlockSpec((H2, 1), lambda i: (0, 0)),
            pl.BlockSpec((1, 1), lambda i: (0, 0)),
        ],
        out_specs=pl.BlockSpec((1, BLK), lambda i: (0, i)),
        out_shape=jax.ShapeDtypeStruct((1, B), jnp.float32),
    )(xt, bt, w1, b1, w2, b2, w3, b3)


def kernel(inputs, emb_tables, bias_tables, W1, b1, W2, b2, W3, b3):
    embT = jnp.transpose(emb_tables, (0, 2, 1)).reshape(K, V)
    biasT = jnp.transpose(bias_tables, (0, 2, 1)).reshape(F, V)
    idxT = inputs.astype(jnp.int32).T

    xt, bt = _sc_gather(embT, biasT, idxT)
    out = _mlp(xt, bt, W1, b1.reshape(1, H1), W2, b2.reshape(1, H2),
               W3, b3.reshape(1, 1))
    return out.reshape(B)


# speedup vs baseline: 37.1342x; 1.1385x over previous
import jax
import jax.numpy as jnp
from jax import lax
from jax.experimental import pallas as pl
from jax.experimental.pallas import tpu as pltpu
from jax.experimental.pallas import tpu_sc as plsc

B, F, V, D = 4096, 26, 100000, 16
H1, H2 = 512, 256

NC, NS = 2, 16
NW = NC * NS
K = F * D
CPW = K // NW

BLK = 512


S0 = 49920
S1 = V - S0


def _sc_body(emb_hbm, bias_hbm, idx_hbm, xt_hbm, bt_hbm,
             buf_a, buf_b, idx_v, out_v, sem_a, sem_b, sem_i):
    wid = lax.axis_index("s") * NC + lax.axis_index("c")
    k0 = wid * CPW
    kend = k0 + CPW

    def fire_a(k):
        pltpu.make_async_copy(emb_hbm.at[k].at[pl.ds(0, S0)], buf_a, sem_a).start()

    def fire_b(k):
        pltpu.make_async_copy(emb_hbm.at[k].at[pl.ds(S0, S1)], buf_b, sem_b).start()

    def wait_a(k):
        pltpu.make_async_copy(emb_hbm.at[k].at[pl.ds(0, S0)], buf_a, sem_a).wait()

    def wait_b(k):
        pltpu.make_async_copy(emb_hbm.at[k].at[pl.ds(S0, S1)], buf_b, sem_b).wait()

    def pass_a(n, _):
        ivec = idx_v[pl.ds(n * 16, 16)]
        m = ivec < S0
        out_v[pl.ds(n * 16, 16)] = plsc.load_gather(buf_a, [ivec], mask=m)
        return _

    def pass_b(n, _):
        ivec = idx_v[pl.ds(n * 16, 16)]
        m = ivec >= S0
        v1 = plsc.load_gather(buf_b, [ivec - S0], mask=m)
        prev = out_v[pl.ds(n * 16, 16)]
        out_v[pl.ds(n * 16, 16)] = jnp.where(m, v1, prev)
        return _

    pltpu.sync_copy(idx_hbm.at[k0 // D], idx_v)
    fire_a(k0)
    fire_b(k0)

    def do_col(k, carry):
        wait_a(k)
        lax.fori_loop(0, B // 16, pass_a, 0, unroll=8)

        @pl.when(k + 1 < kend)
        def _():
            fire_a(k + 1)

        wait_b(k)
        lax.fori_loop(0, B // 16, pass_b, 0, unroll=8)

        @pl.when(k + 1 < kend)
        def _():
            fire_b(k + 1)

        pltpu.sync_copy(out_v, xt_hbm.at[k])

        @pl.when(jnp.logical_and(k + 1 < kend, lax.rem(k + 1, D) == 0))
        def _():
            pltpu.sync_copy(idx_hbm.at[(k + 1) // D], idx_v)

        return carry

    lax.fori_loop(k0, kend, do_col, 0)

    @pl.when(wid < F)
    def _():
        pltpu.sync_copy(idx_hbm.at[wid], idx_v)
        pltpu.make_async_copy(bias_hbm.at[wid].at[pl.ds(0, S0)], buf_a, sem_a).start()
        pltpu.make_async_copy(bias_hbm.at[wid].at[pl.ds(S0, S1)], buf_b, sem_b).start()
        pltpu.make_async_copy(bias_hbm.at[wid].at[pl.ds(0, S0)], buf_a, sem_a).wait()
        lax.fori_loop(0, B // 16, pass_a, 0, unroll=8)
        pltpu.make_async_copy(bias_hbm.at[wid].at[pl.ds(S0, S1)], buf_b, sem_b).wait()
        lax.fori_loop(0, B // 16, pass_b, 0, unroll=8)
        pltpu.sync_copy(out_v, bt_hbm.at[wid])


def _sc_gather(embT, biasT, idxT):
    mesh = plsc.VectorSubcoreMesh(core_axis_name="c", subcore_axis_name="s")
    return pl.kernel(
        _sc_body,
        out_type=(
            jax.ShapeDtypeStruct((K, B), jnp.float32),
            jax.ShapeDtypeStruct((F, B), jnp.float32),
        ),
        mesh=mesh,
        compiler_params=pltpu.CompilerParams(needs_layout_passes=False),
        scratch_types=[
            pltpu.VMEM((S0,), jnp.float32),
            pltpu.VMEM((S1,), jnp.float32),
            pltpu.VMEM((B,), jnp.int32),
            pltpu.VMEM((B,), jnp.float32),
            pltpu.SemaphoreType.DMA,
            pltpu.SemaphoreType.DMA,
            pltpu.SemaphoreType.DMA,
        ],
    )(embT, biasT, idxT)


def _mlp_body(xt_ref, bt_ref, w1_ref, b1_ref, w2_ref, b2_ref, w3_ref, b3_ref,
              out_ref):
    xt = xt_ref[...]
    h = jnp.maximum(
        lax.dot_general(xt, w1_ref[...], (((0,), (0,)), ((), ())),
                        preferred_element_type=jnp.float32)
        + b1_ref[...], 0.0)
    h = jnp.maximum(
        jnp.dot(h, w2_ref[...], preferred_element_type=jnp.float32)
        + b2_ref[...], 0.0)
    o = lax.dot_general(w3_ref[...], h, (((0,), (1,)), ((), ())),
                        preferred_element_type=jnp.float32)
    bsum = jnp.sum(bt_ref[...], axis=0, keepdims=True)
    out_ref[...] = jax.nn.sigmoid(o + bsum + b3_ref[...])


def _mlp(xt, bt, w1, b1, w2, b2, w3, b3):
    return pl.pallas_call(
        _mlp_body,
        grid=(B // BLK,),
        in_specs=[
            pl.BlockSpec((K, BLK), lambda i: (0, i)),
            pl.BlockSpec((F, BLK), lambda i: (0, i)),
            pl.BlockSpec((K, H1), lambda i: (0, 0)),
            pl.BlockSpec((1, H1), lambda i: (0, 0)),
            pl.BlockSpec((H1, H2), lambda i: (0, 0)),
            pl.BlockSpec((1, H2), lambda i: (0, 0)),
            pl.BlockSpec((H2, 1), lambda i: (0, 0)),
            pl.BlockSpec((1, 1), lambda i: (0, 0)),
        ],
        out_specs=pl.BlockSpec((1, BLK), lambda i: (0, i)),
        out_shape=jax.ShapeDtypeStruct((1, B), jnp.float32),
    )(xt, bt, w1, b1, w2, b2, w3, b3)


def kernel(inputs, emb_tables, bias_tables, W1, b1, W2, b2, W3, b3):
    embT = jnp.transpose(emb_tables, (0, 2, 1)).reshape(K, V)
    biasT = jnp.transpose(bias_tables, (0, 2, 1)).reshape(F, V)
    idxT = inputs.astype(jnp.int32).T

    xt, bt = _sc_gather(embT, biasT, idxT)
    out = _mlp(xt, bt, W1, b1.reshape(1, H1), W2, b2.reshape(1, H2),
               W3, b3.reshape(1, 1))
    return out.reshape(B)


# speedup vs baseline: 39.4666x; 1.0628x over previous
import jax
import jax.numpy as jnp
from jax import lax
from jax.experimental import pallas as pl
from jax.experimental.pallas import tpu as pltpu
from jax.experimental.pallas import tpu_sc as plsc

B, F, V, D = 4096, 26, 100000, 16
H1, H2 = 512, 256

NC, NS = 2, 16
NW = NC * NS
K = F * D
CPW = K // NW

BLK = 1024


S0 = 49920
S1 = V - S0


def _sc_body(emb_hbm, bias_hbm, idx_hbm, xt_hbm, bt_hbm,
             buf_a, buf_b, idx_v, out_v, sem_a, sem_b, sem_i):
    wid = lax.axis_index("s") * NC + lax.axis_index("c")
    k0 = wid * CPW
    kend = k0 + CPW

    def fire_a(k):
        pltpu.make_async_copy(emb_hbm.at[k].at[pl.ds(0, S0)], buf_a, sem_a).start()

    def fire_b(k):
        pltpu.make_async_copy(emb_hbm.at[k].at[pl.ds(S0, S1)], buf_b, sem_b).start()

    def wait_a(k):
        pltpu.make_async_copy(emb_hbm.at[k].at[pl.ds(0, S0)], buf_a, sem_a).wait()

    def wait_b(k):
        pltpu.make_async_copy(emb_hbm.at[k].at[pl.ds(S0, S1)], buf_b, sem_b).wait()

    def pass_a(n, _):
        ivec = idx_v[pl.ds(n * 16, 16)]
        m = ivec < S0
        out_v[pl.ds(n * 16, 16)] = plsc.load_gather(buf_a, [ivec], mask=m)
        return _

    def pass_b(n, _):
        ivec = idx_v[pl.ds(n * 16, 16)]
        m = ivec >= S0
        v1 = plsc.load_gather(buf_b, [ivec - S0], mask=m)
        prev = out_v[pl.ds(n * 16, 16)]
        out_v[pl.ds(n * 16, 16)] = jnp.where(m, v1, prev)
        return _

    pltpu.sync_copy(idx_hbm.at[k0 // D], idx_v)
    fire_a(k0)
    fire_b(k0)

    def do_col(k, carry):
        wait_a(k)
        lax.fori_loop(0, B // 16, pass_a, 0, unroll=8)

        @pl.when(k + 1 < kend)
        def _():
            fire_a(k + 1)

        @pl.when(jnp.logical_and(k + 1 == kend, wid < F))
        def _():
            pltpu.make_async_copy(bias_hbm.at[wid].at[pl.ds(0, S0)], buf_a, sem_a).start()

        wait_b(k)
        lax.fori_loop(0, B // 16, pass_b, 0, unroll=8)

        @pl.when(k + 1 < kend)
        def _():
            fire_b(k + 1)

        @pl.when(jnp.logical_and(k + 1 == kend, wid < F))
        def _():
            pltpu.make_async_copy(bias_hbm.at[wid].at[pl.ds(S0, S1)], buf_b, sem_b).start()

        pltpu.sync_copy(out_v, xt_hbm.at[k])

        @pl.when(jnp.logical_and(k + 1 < kend, lax.rem(k + 1, D) == 0))
        def _():
            pltpu.sync_copy(idx_hbm.at[(k + 1) // D], idx_v)

        return carry

    lax.fori_loop(k0, kend, do_col, 0)

    @pl.when(wid < F)
    def _():
        pltpu.sync_copy(idx_hbm.at[wid], idx_v)
        pltpu.make_async_copy(bias_hbm.at[wid].at[pl.ds(0, S0)], buf_a, sem_a).wait()
        lax.fori_loop(0, B // 16, pass_a, 0, unroll=8)
        pltpu.make_async_copy(bias_hbm.at[wid].at[pl.ds(S0, S1)], buf_b, sem_b).wait()
        lax.fori_loop(0, B // 16, pass_b, 0, unroll=8)
        pltpu.sync_copy(out_v, bt_hbm.at[wid])


def _sc_gather(embT, biasT, idxT):
    mesh = plsc.VectorSubcoreMesh(core_axis_name="c", subcore_axis_name="s")
    return pl.kernel(
        _sc_body,
        out_type=(
            jax.ShapeDtypeStruct((K, B), jnp.float32),
            jax.ShapeDtypeStruct((F, B), jnp.float32),
        ),
        mesh=mesh,
        compiler_params=pltpu.CompilerParams(needs_layout_passes=False),
        scratch_types=[
            pltpu.VMEM((S0,), jnp.float32),
            pltpu.VMEM((S1,), jnp.float32),
            pltpu.VMEM((B,), jnp.int32),
            pltpu.VMEM((B,), jnp.float32),
            pltpu.SemaphoreType.DMA,
            pltpu.SemaphoreType.DMA,
            pltpu.SemaphoreType.DMA,
        ],
    )(embT, biasT, idxT)


def _mlp_body(xt_ref, bt_ref, w1_ref, b1_ref, w2_ref, b2_ref, w3_ref, b3_ref,
              out_ref):
    xt = xt_ref[...]
    h = jnp.maximum(
        lax.dot_general(xt, w1_ref[...], (((0,), (0,)), ((), ())),
                        preferred_element_type=jnp.float32)
        + b1_ref[...], 0.0)
    h = jnp.maximum(
        jnp.dot(h, w2_ref[...], preferred_element_type=jnp.float32)
        + b2_ref[...], 0.0)
    o = lax.dot_general(w3_ref[...], h, (((0,), (1,)), ((), ())),
                        preferred_element_type=jnp.float32)
    bsum = jnp.sum(bt_ref[...], axis=0, keepdims=True)
    out_ref[...] = jax.nn.sigmoid(o + bsum + b3_ref[...])


def _mlp(xt, bt, w1, b1, w2, b2, w3, b3):
    return pl.pallas_call(
        _mlp_body,
        grid=(B // BLK,),
        in_specs=[
            pl.BlockSpec((K, BLK), lambda i: (0, i)),
            pl.BlockSpec((F, BLK), lambda i: (0, i)),
            pl.BlockSpec((K, H1), lambda i: (0, 0)),
            pl.BlockSpec((1, H1), lambda i: (0, 0)),
            pl.BlockSpec((H1, H2), lambda i: (0, 0)),
            pl.BlockSpec((1, H2), lambda i: (0, 0)),
            pl.BlockSpec((H2, 1), lambda i: (0, 0)),
            pl.BlockSpec((1, 1), lambda i: (0, 0)),
        ],
        out_specs=pl.BlockSpec((1, BLK), lambda i: (0, i)),
        out_shape=jax.ShapeDtypeStruct((1, B), jnp.float32),
    )(xt, bt, w1, b1, w2, b2, w3, b3)


def kernel(inputs, emb_tables, bias_tables, W1, b1, W2, b2, W3, b3):
    embT = jnp.transpose(emb_tables, (0, 2, 1)).reshape(K, V)
    biasT = jnp.transpose(bias_tables, (0, 2, 1)).reshape(F, V)
    idxT = inputs.astype(jnp.int32).T

    xt, bt = _sc_gather(embT, biasT, idxT)
    out = _mlp(xt, bt, W1, b1.reshape(1, H1), W2, b2.reshape(1, H2),
               W3, b3.reshape(1, 1))
    return out.reshape(B)
